# baseline (speedup 1.0000x reference)
import functools

import jax
import jax.numpy as jnp
from jax import lax
from jax.experimental import pallas as pl
from jax.experimental.pallas import tpu as pltpu
from jax.experimental.pallas import tpu_sc as plsc

N_NODES = 10000
N_EDGES = 160000
INPUT_DIM = 256
HIDDEN_DIM = 128

NC, NS, L = 2, 16, 16
NW = NC * NS
CHUNK = 128
EDGES_PAD = 163840
N_CHUNKS = EDGES_PAD // (NW * CHUNK)
N_PAD = 10240
ROWS_PER_TILE = N_PAD // NS


def _sc_spmm(support, src, dst, w):
    mesh = plsc.VectorSubcoreMesh(core_axis_name="c", subcore_axis_name="s")

    @functools.partial(
        pl.kernel,
        out_type=jax.ShapeDtypeStruct((NC, N_PAD, HIDDEN_DIM), jnp.float32),
        mesh=mesh,
        scratch_types=[
            pltpu.VMEM((N_CHUNKS, CHUNK), jnp.int32),
            pltpu.VMEM((N_CHUNKS, CHUNK), jnp.int32),
            pltpu.VMEM((N_CHUNKS, CHUNK), jnp.float32),
            pltpu.VMEM((CHUNK, HIDDEN_DIM), jnp.float32),
            pltpu.VMEM((CHUNK, HIDDEN_DIM), jnp.float32),
            pltpu.VMEM_SHARED((N_PAD, HIDDEN_DIM), jnp.float32),
            pltpu.SemaphoreType.DMA,
            pltpu.SemaphoreType.DMA,
            pltpu.SemaphoreType.DMA,
            pltpu.SemaphoreType.DMA,
        ],
    )
    def spmm(sup_hbm, src_hbm, dst_hbm, w_hbm, out_hbm,
             src_v, dst_v, w_v, r0buf, r1buf, acc_sh,
             gsem0, gsem1, ssem0, ssem1):
        c = lax.axis_index("c")
        s = lax.axis_index("s")
        wid = c * NS + s
        rbuf = (r0buf, r1buf)
        gsem = (gsem0, gsem1)
        ssem = (ssem0, ssem1)

        pltpu.sync_copy(src_hbm.at[wid], src_v)
        pltpu.sync_copy(dst_hbm.at[wid], dst_v)
        pltpu.sync_copy(w_hbm.at[wid], w_v)

        zero = jnp.zeros((L,), jnp.float32)

        @pl.loop(0, CHUNK)
        def _zrow(r):
            for cs in range(HIDDEN_DIM // L):
                r0buf[r, pl.ds(cs * L, L)] = zero

        @pl.loop(0, ROWS_PER_TILE // CHUNK)
        def _zcopy(k):
            pltpu.sync_copy(r0buf.at[pl.ds(0, CHUNK)],
                            acc_sh.at[pl.ds(s * ROWS_PER_TILE + k * CHUNK,
                                            CHUNK)])

        plsc.subcore_barrier()

        pltpu.async_copy(sup_hbm.at[src_v.at[0]], r0buf, gsem0)
        pltpu.async_copy(sup_hbm.at[src_v.at[1]], r1buf, gsem1)

        @pl.loop(0, N_CHUNKS, step=2)
        def _chunk(j):
            for b in range(2):
                jj = j + b
                pltpu.make_async_copy(sup_hbm.at[src_v.at[jj]],
                                      rbuf[b], gsem[b]).wait()

                @pl.loop(0, CHUNK // L)
                def _grp(g):
                    wv = w_v[jj, pl.ds(g * L, L)]
                    for e in range(L):
                        wsc = wv[e]
                        for cs in range(HIDDEN_DIM // L):
                            sl = pl.ds(cs * L, L)
                            rbuf[b][g * L + e, sl] = rbuf[b][g * L + e, sl] * wsc

                pltpu.async_copy(rbuf[b], acc_sh.at[dst_v.at[jj]],
                                 ssem[b], add=True)
                pltpu.make_async_copy(rbuf[b], acc_sh.at[dst_v.at[jj]],
                                      ssem[b]).wait()

                @pl.when(jj + 2 < N_CHUNKS)
                def _next_gather():
                    pltpu.async_copy(sup_hbm.at[src_v.at[jj + 2]],
                                     rbuf[b], gsem[b])

        plsc.subcore_barrier()

        @pl.loop(0, ROWS_PER_TILE // CHUNK)
        def _wb(k):
            r0 = s * ROWS_PER_TILE + k * CHUNK
            pltpu.sync_copy(acc_sh.at[pl.ds(r0, CHUNK)],
                            r0buf.at[pl.ds(0, CHUNK)])
            pltpu.sync_copy(r0buf.at[pl.ds(0, CHUNK)],
                            out_hbm.at[c, pl.ds(r0, CHUNK)])

    return spmm(support, src, dst, w)


def _mm_encode(fea, W_enc):
    BM = 1000

    def body(x_ref, w_ref, o_ref):
        o_ref[...] = jnp.dot(x_ref[...], w_ref[...],
                             preferred_element_type=jnp.float32)

    return pl.pallas_call(
        body,
        grid=(N_NODES // BM,),
        in_specs=[pl.BlockSpec((BM, INPUT_DIM), lambda i: (i, 0)),
                  pl.BlockSpec((INPUT_DIM, HIDDEN_DIM), lambda i: (0, 0))],
        out_specs=pl.BlockSpec((BM, HIDDEN_DIM), lambda i: (i, 0)),
        out_shape=jax.ShapeDtypeStruct((N_NODES, HIDDEN_DIM), jnp.float32),
    )(fea, W_enc)


def _mm_decode(h2, b_enc, W_dec, b_dec):
    BM = 1000

    def body(h_ref, be_ref, w_ref, bd_ref, o_ref):
        h = h_ref[0] + h_ref[1] + be_ref[...]
        o_ref[...] = jnp.dot(h, w_ref[...],
                             preferred_element_type=jnp.float32) + bd_ref[...]

    return pl.pallas_call(
        body,
        grid=(N_NODES // BM,),
        in_specs=[pl.BlockSpec((NC, BM, HIDDEN_DIM), lambda i: (0, i, 0)),
                  pl.BlockSpec((1, HIDDEN_DIM), lambda i: (0, 0)),
                  pl.BlockSpec((HIDDEN_DIM, INPUT_DIM), lambda i: (0, 0)),
                  pl.BlockSpec((1, INPUT_DIM), lambda i: (0, 0))],
        out_specs=pl.BlockSpec((BM, INPUT_DIM), lambda i: (i, 0)),
        out_shape=jax.ShapeDtypeStruct((N_NODES, INPUT_DIM), jnp.float32),
    )(h2, b_enc.reshape(1, HIDDEN_DIM), W_dec, b_dec.reshape(1, INPUT_DIM))


def kernel(fea, edge_index, edge_weight, W_enc, b_enc, W_dec, b_dec):
    src = edge_index[0].astype(jnp.int32)
    dst = edge_index[1].astype(jnp.int32)
    pad = EDGES_PAD - N_EDGES
    src = jnp.concatenate([src, jnp.zeros((pad,), jnp.int32)])
    dst = jnp.concatenate([dst, jnp.zeros((pad,), jnp.int32)])
    w = jnp.concatenate([edge_weight.astype(jnp.float32),
                         jnp.zeros((pad,), jnp.float32)])
    src = src.reshape(NW, N_CHUNKS, CHUNK)
    dst = dst.reshape(NW, N_CHUNKS, CHUNK)
    w = w.reshape(NW, N_CHUNKS, CHUNK)

    support = _mm_encode(fea, W_enc)
    h2 = _sc_spmm(support, src, dst, w)
    return _mm_decode(h2, b_enc, W_dec, b_dec)


# speedup vs baseline: 1.0953x; 1.0953x over previous
import functools

import jax
import jax.numpy as jnp
from jax import lax
from jax.experimental import pallas as pl
from jax.experimental.pallas import tpu as pltpu
from jax.experimental.pallas import tpu_sc as plsc

N_NODES = 10000
N_EDGES = 160000
INPUT_DIM = 256
HIDDEN_DIM = 128

NC, NS, L = 2, 16, 16
NW = NC * NS
CHUNK = 128
TOT_CHUNKS = 1280
N0_A, N0_B = 32, 32
N1_A, N1_B = 8, 8
CORE1_BASE = NS * (N0_A + N0_B)
CH_PAD = 1304
EDGES_PAD = CH_PAD * CHUNK
STAGE = N0_A
N_PAD = 10240
ROWS_PER_TILE = N_PAD // NS


def _sc_spmm(support, src, dst, w):
    mesh = plsc.VectorSubcoreMesh(core_axis_name="c", subcore_axis_name="s")

    @functools.partial(
        pl.kernel,
        out_type=jax.ShapeDtypeStruct((NC, N_PAD, HIDDEN_DIM), jnp.float32),
        mesh=mesh,
        scratch_types=[
            pltpu.VMEM((STAGE, CHUNK), jnp.int32),
            pltpu.VMEM((STAGE, CHUNK), jnp.int32),
            pltpu.VMEM((STAGE, CHUNK), jnp.float32),
            pltpu.VMEM((CHUNK, HIDDEN_DIM), jnp.float32),
            pltpu.VMEM((CHUNK, HIDDEN_DIM), jnp.float32),
            pltpu.VMEM_SHARED((N_PAD, HIDDEN_DIM), jnp.float32),
            pltpu.SemaphoreType.DMA,
            pltpu.SemaphoreType.DMA,
            pltpu.SemaphoreType.DMA,
            pltpu.SemaphoreType.DMA,
        ],
    )
    def spmm(sup_hbm, src_hbm, dst_hbm, w_hbm, out_hbm,
             src_v, dst_v, w_v, r0buf, r1buf, acc_sh,
             gsem0, gsem1, ssem0, ssem1):
        c = lax.axis_index("c")
        s = lax.axis_index("s")
        rbuf = (r0buf, r1buf)
        gsem = (gsem0, gsem1)
        ssem = (ssem0, ssem1)

        base = lax.select(c == 0, s * (N0_A + N0_B),
                          CORE1_BASE + s * (N1_A + N1_B))
        m_a = lax.select(c == 0, N0_A, N1_A)
        m_b = lax.select(c == 0, N0_B, N1_B)

        zero = jnp.zeros((L,), jnp.float32)

        @pl.loop(0, CHUNK)
        def _zrow(r):
            for cs in range(HIDDEN_DIM // L):
                r0buf[r, pl.ds(cs * L, L)] = zero

        @pl.loop(0, ROWS_PER_TILE // CHUNK)
        def _zcopy(k):
            pltpu.sync_copy(r0buf.at[pl.ds(0, CHUNK)],
                            acc_sh.at[pl.ds(s * ROWS_PER_TILE + k * CHUNK,
                                            CHUNK)])

        plsc.subcore_barrier()

        @pl.loop(0, 2)
        def _stage(stg):
            m = lax.select(stg == 0, m_a, m_b)
            sbase = base + lax.select(stg == 0, 0, m_a)

            pltpu.sync_copy(src_hbm.at[pl.ds(sbase, STAGE)], src_v)
            pltpu.sync_copy(dst_hbm.at[pl.ds(sbase, STAGE)], dst_v)
            pltpu.sync_copy(w_hbm.at[pl.ds(sbase, STAGE)], w_v)

            pltpu.async_copy(sup_hbm.at[src_v.at[0]], r0buf, gsem0)
            pltpu.async_copy(sup_hbm.at[src_v.at[1]], r1buf, gsem1)

            @pl.loop(0, m, step=2)
            def _chunk(j):
                for b in range(2):
                    jj = j + b
                    pltpu.make_async_copy(sup_hbm.at[src_v.at[jj]],
                                          rbuf[b], gsem[b]).wait()

                    @pl.loop(0, CHUNK // L)
                    def _grp(g):
                        wv = w_v[jj, pl.ds(g * L, L)]
                        for e in range(L):
                            wsc = wv[e]
                            for cs in range(HIDDEN_DIM // L):
                                sl = pl.ds(cs * L, L)
                                rbuf[b][g * L + e, sl] = \
                                    rbuf[b][g * L + e, sl] * wsc

                    pltpu.async_copy(rbuf[b], acc_sh.at[dst_v.at[jj]],
                                     ssem[b], add=True)
                    pltpu.make_async_copy(rbuf[b], acc_sh.at[dst_v.at[jj]],
                                          ssem[b]).wait()

                    @pl.when(jj + 2 < m)
                    def _next_gather():
                        pltpu.async_copy(sup_hbm.at[src_v.at[jj + 2]],
                                         rbuf[b], gsem[b])

        plsc.subcore_barrier()

        @pl.loop(0, ROWS_PER_TILE // CHUNK)
        def _wb(k):
            r0 = s * ROWS_PER_TILE + k * CHUNK
            pltpu.sync_copy(acc_sh.at[pl.ds(r0, CHUNK)],
                            r0buf.at[pl.ds(0, CHUNK)])
            pltpu.sync_copy(r0buf.at[pl.ds(0, CHUNK)],
                            out_hbm.at[c, pl.ds(r0, CHUNK)])

    return spmm(support, src, dst, w)


def _mm_encode(fea, W_enc):
    BM = 1000

    def body(x_ref, w_ref, o_ref):
        o_ref[...] = jnp.dot(x_ref[...], w_ref[...],
                             preferred_element_type=jnp.float32)

    return pl.pallas_call(
        body,
        grid=(N_NODES // BM,),
        in_specs=[pl.BlockSpec((BM, INPUT_DIM), lambda i: (i, 0)),
                  pl.BlockSpec((INPUT_DIM, HIDDEN_DIM), lambda i: (0, 0))],
        out_specs=pl.BlockSpec((BM, HIDDEN_DIM), lambda i: (i, 0)),
        out_shape=jax.ShapeDtypeStruct((N_NODES, HIDDEN_DIM), jnp.float32),
    )(fea, W_enc)


def _mm_decode(h2, b_enc, W_dec, b_dec):
    BM = 1000

    def body(h_ref, be_ref, w_ref, bd_ref, o_ref):
        h = h_ref[0] + h_ref[1] + be_ref[...]
        o_ref[...] = jnp.dot(h, w_ref[...],
                             preferred_element_type=jnp.float32) + bd_ref[...]

    return pl.pallas_call(
        body,
        grid=(N_NODES // BM,),
        in_specs=[pl.BlockSpec((NC, BM, HIDDEN_DIM), lambda i: (0, i, 0)),
                  pl.BlockSpec((1, HIDDEN_DIM), lambda i: (0, 0)),
                  pl.BlockSpec((HIDDEN_DIM, INPUT_DIM), lambda i: (0, 0)),
                  pl.BlockSpec((1, INPUT_DIM), lambda i: (0, 0))],
        out_specs=pl.BlockSpec((BM, INPUT_DIM), lambda i: (i, 0)),
        out_shape=jax.ShapeDtypeStruct((N_NODES, INPUT_DIM), jnp.float32),
    )(h2, b_enc.reshape(1, HIDDEN_DIM), W_dec, b_dec.reshape(1, INPUT_DIM))


def kernel(fea, edge_index, edge_weight, W_enc, b_enc, W_dec, b_dec):
    src = edge_index[0].astype(jnp.int32)
    dst = edge_index[1].astype(jnp.int32)
    pad = EDGES_PAD - N_EDGES
    src = jnp.concatenate([src, jnp.zeros((pad,), jnp.int32)])
    dst = jnp.concatenate([dst, jnp.zeros((pad,), jnp.int32)])
    w = jnp.concatenate([edge_weight.astype(jnp.float32),
                         jnp.zeros((pad,), jnp.float32)])
    src = src.reshape(CH_PAD, CHUNK)
    dst = dst.reshape(CH_PAD, CHUNK)
    w = w.reshape(CH_PAD, CHUNK)

    support = _mm_encode(fea, W_enc)
    h2 = _sc_spmm(support, src, dst, w)
    return _mm_decode(h2, b_enc, W_dec, b_dec)


# speedup vs baseline: 1.0991x; 1.0034x over previous
import functools

import jax
import jax.numpy as jnp
from jax import lax
from jax.experimental import pallas as pl
from jax.experimental.pallas import tpu as pltpu
from jax.experimental.pallas import tpu_sc as plsc

N_NODES = 10000
N_EDGES = 160000
INPUT_DIM = 256
HIDDEN_DIM = 128

NC, NS, L = 2, 16, 16
NW = NC * NS
CHUNK = 128
TOT_CHUNKS = 1280
N0_A, N0_B = 32, 32
N1_A, N1_B = 8, 8
CORE1_BASE = NS * (N0_A + N0_B)
CH_PAD = 1304
EDGES_PAD = CH_PAD * CHUNK
STAGE = N0_A
N_PAD = 10240
ROWS_PER_TILE = N_PAD // NS


def _sc_spmm(support, src, dst, w):
    mesh = plsc.VectorSubcoreMesh(core_axis_name="c", subcore_axis_name="s")

    @functools.partial(
        pl.kernel,
        out_type=jax.ShapeDtypeStruct((NC, N_PAD, HIDDEN_DIM), jnp.float32),
        mesh=mesh,
        scratch_types=[
            pltpu.VMEM((STAGE, CHUNK), jnp.int32),
            pltpu.VMEM((STAGE, CHUNK), jnp.int32),
            pltpu.VMEM((STAGE, CHUNK), jnp.float32),
            pltpu.VMEM((CHUNK, HIDDEN_DIM), jnp.float32),
            pltpu.VMEM((CHUNK, HIDDEN_DIM), jnp.float32),
            pltpu.VMEM_SHARED((N_PAD, HIDDEN_DIM), jnp.float32),
            pltpu.SemaphoreType.DMA,
            pltpu.SemaphoreType.DMA,
            pltpu.SemaphoreType.DMA,
            pltpu.SemaphoreType.DMA,
        ],
    )
    def spmm(sup_hbm, src_hbm, dst_hbm, w_hbm, out_hbm,
             src_v, dst_v, w_v, r0buf, r1buf, acc_sh,
             gsem0, gsem1, ssem0, ssem1):
        c = lax.axis_index("c")
        s = lax.axis_index("s")
        rbuf = (r0buf, r1buf)
        gsem = (gsem0, gsem1)
        ssem = (ssem0, ssem1)

        base = lax.select(c == 1, s * (N0_A + N0_B),
                          CORE1_BASE + s * (N1_A + N1_B))
        m_a = lax.select(c == 1, N0_A, N1_A)
        m_b = lax.select(c == 1, N0_B, N1_B)

        zero = jnp.zeros((L,), jnp.float32)

        @pl.loop(0, CHUNK)
        def _zrow(r):
            for cs in range(HIDDEN_DIM // L):
                r0buf[r, pl.ds(cs * L, L)] = zero

        @pl.loop(0, ROWS_PER_TILE // CHUNK)
        def _zcopy(k):
            pltpu.sync_copy(r0buf.at[pl.ds(0, CHUNK)],
                            acc_sh.at[pl.ds(s * ROWS_PER_TILE + k * CHUNK,
                                            CHUNK)])

        plsc.subcore_barrier()

        @pl.loop(0, 2)
        def _stage(stg):
            m = lax.select(stg == 0, m_a, m_b)
            sbase = base + lax.select(stg == 0, 0, m_a)

            pltpu.sync_copy(src_hbm.at[pl.ds(sbase, STAGE)], src_v)
            pltpu.sync_copy(dst_hbm.at[pl.ds(sbase, STAGE)], dst_v)
            pltpu.sync_copy(w_hbm.at[pl.ds(sbase, STAGE)], w_v)

            pltpu.async_copy(sup_hbm.at[src_v.at[0]], r0buf, gsem0)
            pltpu.async_copy(sup_hbm.at[src_v.at[1]], r1buf, gsem1)

            @pl.loop(0, m, step=2)
            def _chunk(j):
                for b in range(2):
                    jj = j + b
                    pltpu.make_async_copy(sup_hbm.at[src_v.at[jj]],
                                          rbuf[b], gsem[b]).wait()

                    @pl.loop(0, CHUNK // L)
                    def _grp(g):
                        wv = w_v[jj, pl.ds(g * L, L)]
                        for e in range(L):
                            wsc = wv[e]
                            for cs in range(HIDDEN_DIM // L):
                                sl = pl.ds(cs * L, L)
                                rbuf[b][g * L + e, sl] = \
                                    rbuf[b][g * L + e, sl] * wsc

                    pltpu.async_copy(rbuf[b], acc_sh.at[dst_v.at[jj]],
                                     ssem[b], add=True)
                    pltpu.make_async_copy(rbuf[b], acc_sh.at[dst_v.at[jj]],
                                          ssem[b]).wait()

                    @pl.when(jj + 2 < m)
                    def _next_gather():
                        pltpu.async_copy(sup_hbm.at[src_v.at[jj + 2]],
                                         rbuf[b], gsem[b])

        plsc.subcore_barrier()

        @pl.loop(0, ROWS_PER_TILE // CHUNK)
        def _wb(k):
            r0 = s * ROWS_PER_TILE + k * CHUNK
            pltpu.sync_copy(acc_sh.at[pl.ds(r0, CHUNK)],
                            r0buf.at[pl.ds(0, CHUNK)])
            pltpu.sync_copy(r0buf.at[pl.ds(0, CHUNK)],
                            out_hbm.at[c, pl.ds(r0, CHUNK)])

    return spmm(support, src, dst, w)


def _mm_encode(fea, W_enc):
    BM = 1000

    def body(x_ref, w_ref, o_ref):
        o_ref[...] = jnp.dot(x_ref[...], w_ref[...],
                             preferred_element_type=jnp.float32)

    return pl.pallas_call(
        body,
        grid=(N_NODES // BM,),
        in_specs=[pl.BlockSpec((BM, INPUT_DIM), lambda i: (i, 0)),
                  pl.BlockSpec((INPUT_DIM, HIDDEN_DIM), lambda i: (0, 0))],
        out_specs=pl.BlockSpec((BM, HIDDEN_DIM), lambda i: (i, 0)),
        out_shape=jax.ShapeDtypeStruct((N_NODES, HIDDEN_DIM), jnp.float32),
    )(fea, W_enc)


def _mm_decode(h2, b_enc, W_dec, b_dec):
    BM = 1000

    def body(h_ref, be_ref, w_ref, bd_ref, o_ref):
        h = h_ref[0] + h_ref[1] + be_ref[...]
        o_ref[...] = jnp.dot(h, w_ref[...],
                             preferred_element_type=jnp.float32) + bd_ref[...]

    return pl.pallas_call(
        body,
        grid=(N_NODES // BM,),
        in_specs=[pl.BlockSpec((NC, BM, HIDDEN_DIM), lambda i: (0, i, 0)),
                  pl.BlockSpec((1, HIDDEN_DIM), lambda i: (0, 0)),
                  pl.BlockSpec((HIDDEN_DIM, INPUT_DIM), lambda i: (0, 0)),
                  pl.BlockSpec((1, INPUT_DIM), lambda i: (0, 0))],
        out_specs=pl.BlockSpec((BM, INPUT_DIM), lambda i: (i, 0)),
        out_shape=jax.ShapeDtypeStruct((N_NODES, INPUT_DIM), jnp.float32),
    )(h2, b_enc.reshape(1, HIDDEN_DIM), W_dec, b_dec.reshape(1, INPUT_DIM))


def kernel(fea, edge_index, edge_weight, W_enc, b_enc, W_dec, b_dec):
    src = edge_index[0].astype(jnp.int32)
    dst = edge_index[1].astype(jnp.int32)
    pad = EDGES_PAD - N_EDGES
    src = jnp.concatenate([src, jnp.zeros((pad,), jnp.int32)])
    dst = jnp.concatenate([dst, jnp.zeros((pad,), jnp.int32)])
    w = jnp.concatenate([edge_weight.astype(jnp.float32),
                         jnp.zeros((pad,), jnp.float32)])
    src = src.reshape(CH_PAD, CHUNK)
    dst = dst.reshape(CH_PAD, CHUNK)
    w = w.reshape(CH_PAD, CHUNK)

    support = _mm_encode(fea, W_enc)
    h2 = _sc_spmm(support, src, dst, w)
    return _mm_decode(h2, b_enc, W_dec, b_dec)
